# baseline (speedup 1.0000x reference)
import functools
import math

import numpy as np
import jax
import jax.numpy as jnp
from jax import lax
from jax.experimental import pallas as pl
from jax.experimental.pallas import tpu as pltpu
from jax.experimental.pallas import tpu_sc as plsc

TEXT_MAX_LEN = 2048
D_MODEL = 768
EPS = 1e-05

_NC = 2
_NS = 16
_NW = _NC * _NS


def _position_encoding_np(length, d_model, min_timescale=1.0, max_timescale=10000.0):
    position = np.arange(length, dtype=np.float32)
    num_timescales = d_model // 2
    log_timescale_increment = math.log(float(max_timescale) / float(min_timescale)) / (
        float(num_timescales) - 1.0
    )
    inv_timescales = min_timescale * np.exp(
        np.arange(num_timescales, dtype=np.float32) * -log_timescale_increment
    )
    scaled_time = position[:, None] * inv_timescales[None, :]
    return np.concatenate(
        [np.sin(scaled_time), np.cos(scaled_time)], axis=1
    ).astype(np.float32)


_POS = _position_encoding_np(TEXT_MAX_LEN, D_MODEL)


def _make_sc_gather(n_tokens, d):
    assert n_tokens % _NW == 0
    per_w = n_tokens // _NW
    n_chunks = 4
    assert per_w % n_chunks == 0
    chunk = per_w // n_chunks

    mesh = plsc.VectorSubcoreMesh(core_axis_name="c", subcore_axis_name="s")

    @functools.partial(
        pl.kernel,
        mesh=mesh,
        out_type=jax.ShapeDtypeStruct((n_tokens, d), jnp.float32),
        scratch_types=[
            pltpu.VMEM((per_w,), jnp.int32),
            pltpu.VMEM((2, chunk, d), jnp.float32),
            pltpu.SemaphoreType.DMA,
            pltpu.SemaphoreType.DMA,
        ],
    )
    def gather_kernel(idx_hbm, table_hbm, out_hbm, idx_v, buf_v, sem0, sem1):
        wid = lax.axis_index("s") * _NC + lax.axis_index("c")
        base = wid * per_w
        pltpu.sync_copy(idx_hbm.at[pl.ds(base, per_w)], idx_v)
        sems = (sem0, sem1)
        cps = [None, None]
        for c in range(n_chunks):
            b = c & 1
            if c >= 2:
                cps[b].wait()
                pltpu.sync_copy(
                    buf_v.at[b], out_hbm.at[pl.ds(base + (c - 2) * chunk, chunk)]
                )
            cps[b] = pltpu.async_copy(
                table_hbm.at[idx_v.at[pl.ds(c * chunk, chunk)]],
                buf_v.at[b],
                sems[b],
            )
        for c in range(n_chunks - 2, n_chunks):
            b = c & 1
            cps[b].wait()
            pltpu.sync_copy(
                buf_v.at[b], out_hbm.at[pl.ds(base + c * chunk, chunk)]
            )

    return gather_kernel


def _tc_addln_body(x_ref, pos_ref, g_ref, b_ref, o_ref):
    x = x_ref[...] + pos_ref[...]
    mean = jnp.mean(x, axis=-1, keepdims=True)
    xc = x - mean
    var = jnp.mean(xc * xc, axis=-1, keepdims=True)
    o_ref[...] = xc * lax.rsqrt(var + EPS) * g_ref[...] + b_ref[...]


def _tc_addln(gathered, pos, gamma, beta):
    n, d = gathered.shape
    l = pos.shape[0]
    blk = 1024
    grid = (n // blk,)
    pos_blocks = l // blk
    return pl.pallas_call(
        _tc_addln_body,
        grid=grid,
        in_specs=[
            pl.BlockSpec((blk, d), lambda g: (g, 0)),
            pl.BlockSpec((blk, d), lambda g: (g % pos_blocks, 0)),
            pl.BlockSpec((1, d), lambda g: (0, 0)),
            pl.BlockSpec((1, d), lambda g: (0, 0)),
        ],
        out_specs=pl.BlockSpec((blk, d), lambda g: (g, 0)),
        out_shape=jax.ShapeDtypeStruct((n, d), jnp.float32),
    )(gathered, pos, gamma, beta)


def _make_sc_fused(n_tokens, d, l):
    assert n_tokens % _NW == 0
    per_w = n_tokens // _NW
    n_chunks = 8
    chunk = per_w // n_chunks
    nj = d // 16
    inv_d = 1.0 / d

    mesh = plsc.VectorSubcoreMesh(core_axis_name="c", subcore_axis_name="s")

    @functools.partial(
        pl.kernel,
        mesh=mesh,
        out_type=jax.ShapeDtypeStruct((n_tokens, d), jnp.float32),
        scratch_types=[
            pltpu.VMEM((per_w,), jnp.int32),
            pltpu.VMEM((2, chunk, d), jnp.float32),
            pltpu.VMEM((2, chunk, d), jnp.float32),
            pltpu.VMEM((d,), jnp.float32),
            pltpu.VMEM((d,), jnp.float32),
            pltpu.SemaphoreType.DMA,
            pltpu.SemaphoreType.DMA,
            pltpu.SemaphoreType.DMA,
            pltpu.SemaphoreType.DMA,
        ],
    )
    def fused(idx_hbm, table_hbm, pos_hbm, g_hbm, b_hbm, out_hbm,
              idx_v, buf_v, pos_v, g_v, b_v, s0, s1, p0, p1):
        wid = lax.axis_index("s") * _NC + lax.axis_index("c")
        base = wid * per_w
        pos_base = lax.rem(base, l)
        pltpu.sync_copy(g_hbm, g_v)
        pltpu.sync_copy(b_hbm, b_v)
        pltpu.sync_copy(idx_hbm.at[pl.ds(base, per_w)], idx_v)
        gsems = (s0, s1)
        psems = (p0, p1)
        gcp = [None, None]
        pcp = [None, None]

        def fire(c):
            bb = c & 1
            gcp[bb] = pltpu.async_copy(
                table_hbm.at[idx_v.at[pl.ds(c * chunk, chunk)]],
                buf_v.at[bb],
                gsems[bb],
            )
            pcp[bb] = pltpu.async_copy(
                pos_hbm.at[pl.ds(pos_base + c * chunk, chunk)],
                pos_v.at[bb],
                psems[bb],
            )

        fire(0)
        for c in range(n_chunks):
            bb = c & 1
            gcp[bb].wait()
            pcp[bb].wait()
            if c + 1 < n_chunks:
                fire(c + 1)
            buf_b = buf_v.at[bb]
            pos_b = pos_v.at[bb]

            lane_iota = lax.iota(jnp.int32, 16)

            def lane_allsum(v):
                for k in (8, 4, 2, 1):
                    p = lax.bitwise_xor(lane_iota, jnp.int32(k))
                    v = v + v.at[p].get(mode="promise_in_bounds")
                return v

            def row_body(r, carry):
                accs = [jnp.zeros((16,), jnp.float32) for _ in range(4)]
                acc2s = [jnp.zeros((16,), jnp.float32) for _ in range(4)]
                for j in range(nj):
                    sl = pl.ds(16 * j, 16)
                    x = buf_b[r, sl] + pos_b[r, sl]
                    buf_b[r, sl] = x
                    accs[j & 3] = accs[j & 3] + x
                    acc2s[j & 3] = acc2s[j & 3] + x * x
                acc = (accs[0] + accs[1]) + (accs[2] + accs[3])
                acc2 = (acc2s[0] + acc2s[1]) + (acc2s[2] + acc2s[3])
                mean = lane_allsum(acc) * inv_d
                var = lane_allsum(acc2) * inv_d - mean * mean
                v = var + EPS
                iv = lax.bitcast_convert_type(v, jnp.int32)
                y = lax.bitcast_convert_type(
                    jnp.full((16,), 0x5F3759DF, jnp.int32)
                    - lax.shift_right_arithmetic(iv, 1),
                    jnp.float32,
                )
                for _ in range(3):
                    y = y * (1.5 - 0.5 * v * y * y)
                for j in range(nj):
                    sl = pl.ds(16 * j, 16)
                    xj = buf_b[r, sl]
                    buf_b[r, sl] = (xj - mean) * y * g_v[sl] + b_v[sl]
                return carry

            lax.fori_loop(0, chunk, row_body, 0)
            pltpu.sync_copy(buf_b, out_hbm.at[pl.ds(base + c * chunk, chunk)])

    return fused


def kernel(inputs, table, ln_gamma, ln_beta):
    b, l = inputs.shape
    _, d = table.shape
    idx = inputs.reshape(-1).astype(jnp.int32)
    pos = jnp.asarray(_POS)
    out = _make_sc_fused(b * l, d, l)(idx, table, pos, ln_gamma, ln_beta)
    return out.reshape(b, l, d)


# speedup vs baseline: 2.3198x; 2.3198x over previous
import functools
import math

import numpy as np
import jax
import jax.numpy as jnp
from jax import lax
from jax.experimental import pallas as pl
from jax.experimental.pallas import tpu as pltpu
from jax.experimental.pallas import tpu_sc as plsc

TEXT_MAX_LEN = 2048
D_MODEL = 768
EPS = 1e-05

_NC = 2
_NS = 16
_NW = _NC * _NS


def _position_encoding_np(length, d_model, min_timescale=1.0, max_timescale=10000.0):
    position = np.arange(length, dtype=np.float32)
    num_timescales = d_model // 2
    log_timescale_increment = math.log(float(max_timescale) / float(min_timescale)) / (
        float(num_timescales) - 1.0
    )
    inv_timescales = min_timescale * np.exp(
        np.arange(num_timescales, dtype=np.float32) * -log_timescale_increment
    )
    scaled_time = position[:, None] * inv_timescales[None, :]
    return np.concatenate(
        [np.sin(scaled_time), np.cos(scaled_time)], axis=1
    ).astype(np.float32)


_POS = _position_encoding_np(TEXT_MAX_LEN, D_MODEL)


def _make_sc_gather(n_tokens, d):
    assert n_tokens % _NW == 0
    per_w = n_tokens // _NW
    n_chunks = 4
    assert per_w % n_chunks == 0
    chunk = per_w // n_chunks

    mesh = plsc.VectorSubcoreMesh(core_axis_name="c", subcore_axis_name="s")

    @functools.partial(
        pl.kernel,
        mesh=mesh,
        out_type=jax.ShapeDtypeStruct((n_tokens, d), jnp.float32),
        scratch_types=[
            pltpu.VMEM((per_w,), jnp.int32),
            pltpu.VMEM((2, chunk, d), jnp.float32),
            pltpu.SemaphoreType.DMA,
            pltpu.SemaphoreType.DMA,
        ],
    )
    def gather_kernel(idx_hbm, table_hbm, out_hbm, idx_v, buf_v, sem0, sem1):
        wid = lax.axis_index("s") * _NC + lax.axis_index("c")
        base = wid * per_w
        pltpu.sync_copy(idx_hbm.at[pl.ds(base, per_w)], idx_v)
        sems = (sem0, sem1)
        cps = [None, None]
        for c in range(n_chunks):
            b = c & 1
            if c >= 2:
                cps[b].wait()
                pltpu.sync_copy(
                    buf_v.at[b], out_hbm.at[pl.ds(base + (c - 2) * chunk, chunk)]
                )
            cps[b] = pltpu.async_copy(
                table_hbm.at[idx_v.at[pl.ds(c * chunk, chunk)]],
                buf_v.at[b],
                sems[b],
            )
        for c in range(n_chunks - 2, n_chunks):
            b = c & 1
            cps[b].wait()
            pltpu.sync_copy(
                buf_v.at[b], out_hbm.at[pl.ds(base + c * chunk, chunk)]
            )

    return gather_kernel


def _tc_addln_body(x_ref, pos_ref, g_ref, b_ref, o_ref):
    x = x_ref[...] + pos_ref[...]
    mean = jnp.mean(x, axis=-1, keepdims=True)
    xc = x - mean
    var = jnp.mean(xc * xc, axis=-1, keepdims=True)
    o_ref[...] = xc * lax.rsqrt(var + EPS) * g_ref[...] + b_ref[...]


def _tc_addln(gathered, pos, gamma, beta):
    n, d = gathered.shape
    l = pos.shape[0]
    blk = 1024
    pos_blocks = l // blk
    batches = n // l
    grid = (pos_blocks, batches)
    return pl.pallas_call(
        _tc_addln_body,
        grid=grid,
        in_specs=[
            pl.BlockSpec((blk, d), lambda p, b: (b * pos_blocks + p, 0)),
            pl.BlockSpec((blk, d), lambda p, b: (p, 0)),
            pl.BlockSpec((1, d), lambda p, b: (0, 0)),
            pl.BlockSpec((1, d), lambda p, b: (0, 0)),
        ],
        out_specs=pl.BlockSpec((blk, d), lambda p, b: (b * pos_blocks + p, 0)),
        out_shape=jax.ShapeDtypeStruct((n, d), jnp.float32),
    )(gathered, pos, gamma, beta)


def _make_sc_fused(n_tokens, d, l):
    assert n_tokens % _NW == 0
    per_w = n_tokens // _NW
    n_chunks = 8
    chunk = per_w // n_chunks
    nj = d // 16
    inv_d = 1.0 / d

    mesh = plsc.VectorSubcoreMesh(core_axis_name="c", subcore_axis_name="s")

    @functools.partial(
        pl.kernel,
        mesh=mesh,
        out_type=jax.ShapeDtypeStruct((n_tokens, d), jnp.float32),
        scratch_types=[
            pltpu.VMEM((per_w,), jnp.int32),
            pltpu.VMEM((2, chunk, d), jnp.float32),
            pltpu.VMEM((2, chunk, d), jnp.float32),
            pltpu.VMEM((d,), jnp.float32),
            pltpu.VMEM((d,), jnp.float32),
            pltpu.SemaphoreType.DMA,
            pltpu.SemaphoreType.DMA,
            pltpu.SemaphoreType.DMA,
            pltpu.SemaphoreType.DMA,
        ],
    )
    def fused(idx_hbm, table_hbm, pos_hbm, g_hbm, b_hbm, out_hbm,
              idx_v, buf_v, pos_v, g_v, b_v, s0, s1, p0, p1):
        wid = lax.axis_index("s") * _NC + lax.axis_index("c")
        base = wid * per_w
        pos_base = lax.rem(base, l)
        pltpu.sync_copy(g_hbm, g_v)
        pltpu.sync_copy(b_hbm, b_v)
        pltpu.sync_copy(idx_hbm.at[pl.ds(base, per_w)], idx_v)
        gsems = (s0, s1)
        psems = (p0, p1)
        gcp = [None, None]
        pcp = [None, None]

        def fire(c):
            bb = c & 1
            gcp[bb] = pltpu.async_copy(
                table_hbm.at[idx_v.at[pl.ds(c * chunk, chunk)]],
                buf_v.at[bb],
                gsems[bb],
            )
            pcp[bb] = pltpu.async_copy(
                pos_hbm.at[pl.ds(pos_base + c * chunk, chunk)],
                pos_v.at[bb],
                psems[bb],
            )

        fire(0)
        for c in range(n_chunks):
            bb = c & 1
            gcp[bb].wait()
            pcp[bb].wait()
            if c + 1 < n_chunks:
                fire(c + 1)
            buf_b = buf_v.at[bb]
            pos_b = pos_v.at[bb]

            lane_iota = lax.iota(jnp.int32, 16)

            def lane_allsum(v):
                for k in (8, 4, 2, 1):
                    p = lax.bitwise_xor(lane_iota, jnp.int32(k))
                    v = v + v.at[p].get(mode="promise_in_bounds")
                return v

            def row_body(r, carry):
                accs = [jnp.zeros((16,), jnp.float32) for _ in range(4)]
                acc2s = [jnp.zeros((16,), jnp.float32) for _ in range(4)]
                for j in range(nj):
                    sl = pl.ds(16 * j, 16)
                    x = buf_b[r, sl] + pos_b[r, sl]
                    buf_b[r, sl] = x
                    accs[j & 3] = accs[j & 3] + x
                    acc2s[j & 3] = acc2s[j & 3] + x * x
                acc = (accs[0] + accs[1]) + (accs[2] + accs[3])
                acc2 = (acc2s[0] + acc2s[1]) + (acc2s[2] + acc2s[3])
                mean = lane_allsum(acc) * inv_d
                var = lane_allsum(acc2) * inv_d - mean * mean
                v = var + EPS
                iv = lax.bitcast_convert_type(v, jnp.int32)
                y = lax.bitcast_convert_type(
                    jnp.full((16,), 0x5F3759DF, jnp.int32)
                    - lax.shift_right_arithmetic(iv, 1),
                    jnp.float32,
                )
                for _ in range(3):
                    y = y * (1.5 - 0.5 * v * y * y)
                for j in range(nj):
                    sl = pl.ds(16 * j, 16)
                    xj = buf_b[r, sl]
                    buf_b[r, sl] = (xj - mean) * y * g_v[sl] + b_v[sl]
                return carry

            lax.fori_loop(0, chunk, row_body, 0)
            pltpu.sync_copy(buf_b, out_hbm.at[pl.ds(base + c * chunk, chunk)])

    return fused


def kernel(inputs, table, ln_gamma, ln_beta):
    b, l = inputs.shape
    _, d = table.shape
    idx = inputs.reshape(-1).astype(jnp.int32)
    gathered = _make_sc_gather(b * l, d)(idx, table)
    pos = jnp.asarray(_POS)
    out = _tc_addln(gathered, pos, ln_gamma.reshape(1, d), ln_beta.reshape(1, d))
    return out.reshape(b, l, d)


# speedup vs baseline: 2.3430x; 1.0100x over previous
import functools
import math

import numpy as np
import jax
import jax.numpy as jnp
from jax import lax
from jax.experimental import pallas as pl
from jax.experimental.pallas import tpu as pltpu
from jax.experimental.pallas import tpu_sc as plsc

TEXT_MAX_LEN = 2048
D_MODEL = 768
EPS = 1e-05

_NC = 2
_NS = 16
_NW = _NC * _NS


def _position_encoding_np(length, d_model, min_timescale=1.0, max_timescale=10000.0):
    position = np.arange(length, dtype=np.float32)
    num_timescales = d_model // 2
    log_timescale_increment = math.log(float(max_timescale) / float(min_timescale)) / (
        float(num_timescales) - 1.0
    )
    inv_timescales = min_timescale * np.exp(
        np.arange(num_timescales, dtype=np.float32) * -log_timescale_increment
    )
    scaled_time = position[:, None] * inv_timescales[None, :]
    return np.concatenate(
        [np.sin(scaled_time), np.cos(scaled_time)], axis=1
    ).astype(np.float32)


_POS = _position_encoding_np(TEXT_MAX_LEN, D_MODEL)


def _make_sc_gather(bsz, seq, d):
    n_tokens = bsz * seq
    assert n_tokens % _NW == 0
    per_w = n_tokens // _NW
    assert seq % per_w == 0
    nbuf = 4
    chunk = 32
    n_chunks = per_w // chunk

    mesh = plsc.VectorSubcoreMesh(core_axis_name="c", subcore_axis_name="s")

    @functools.partial(
        pl.kernel,
        mesh=mesh,
        out_type=jax.ShapeDtypeStruct((n_tokens, d), jnp.float32),
        scratch_types=[
            pltpu.VMEM((per_w,), jnp.int32),
            pltpu.VMEM((nbuf, chunk, d), jnp.float32),
        ]
        + [pltpu.SemaphoreType.DMA] * (2 * nbuf),
    )
    def gather_kernel(idx_hbm, table_hbm, out_hbm, idx_v, buf_v, *sems):
        gsem = sems[:nbuf]
        osem = sems[nbuf:]
        wid = lax.axis_index("s") * _NC + lax.axis_index("c")
        base = wid * per_w
        row = wid // (seq // per_w)
        col = lax.rem(base, seq)
        pltpu.sync_copy(idx_hbm.at[row, pl.ds(col, per_w)], idx_v)

        gcp = [None] * nbuf
        ocp = [None] * nbuf

        def fire(c):
            b = c % nbuf
            gcp[b] = pltpu.async_copy(
                table_hbm.at[idx_v.at[pl.ds(c * chunk, chunk)]],
                buf_v.at[b],
                gsem[b],
            )

        for c in range(min(nbuf, n_chunks)):
            fire(c)
        for c in range(n_chunks):
            b = c % nbuf
            gcp[b].wait()
            ocp[b] = pltpu.async_copy(
                buf_v.at[b], out_hbm.at[pl.ds(base + c * chunk, chunk)], osem[b]
            )
            nxt = c + nbuf
            if nxt < n_chunks:
                ocp[b].wait()
                fire(nxt)
        for c in range(max(0, n_chunks - nbuf), n_chunks):
            ocp[c % nbuf].wait()

    return gather_kernel


def _tc_addln_body(x_ref, pos_ref, g_ref, b_ref, o_ref):
    x = x_ref[0] + pos_ref[...]
    mean = jnp.mean(x, axis=-1, keepdims=True)
    xc = x - mean
    var = jnp.mean(xc * xc, axis=-1, keepdims=True)
    o_ref[0] = xc * lax.rsqrt(var + EPS) * g_ref[...] + b_ref[...]


def _tc_addln(gathered, pos, gamma, beta, bsz, seq):
    n, d = gathered.shape
    grid = (bsz,)
    return pl.pallas_call(
        _tc_addln_body,
        grid=grid,
        in_specs=[
            pl.BlockSpec((1, seq, d), lambda b: (b, 0, 0)),
            pl.BlockSpec((seq, d), lambda b: (0, 0)),
            pl.BlockSpec((1, d), lambda b: (0, 0)),
            pl.BlockSpec((1, d), lambda b: (0, 0)),
        ],
        out_specs=pl.BlockSpec((1, seq, d), lambda b: (b, 0, 0)),
        out_shape=jax.ShapeDtypeStruct((bsz, seq, d), jnp.float32),
    )(gathered.reshape(bsz, seq, d), pos, gamma, beta)


def kernel(inputs, table, ln_gamma, ln_beta):
    bsz, seq = inputs.shape
    _, d = table.shape
    idx = inputs.astype(jnp.int32)
    gathered = _make_sc_gather(bsz, seq, d)(idx, table)
    pos = jnp.asarray(_POS)
    return _tc_addln(
        gathered, pos, ln_gamma.reshape(1, d), ln_beta.reshape(1, d), bsz, seq
    )
